# baseline (speedup 1.0000x reference)
import functools

import jax
import jax.numpy as jnp
from jax import lax
from jax.experimental import pallas as pl
from jax.experimental.pallas import tpu as pltpu
from jax.experimental.pallas import tpu_sc as plsc

N_POINTS = 2_000_000
CHUNK = 8_000
N_CHUNKS = N_POINTS // CHUNK
NW = 32
BASE_ITERS, EXTRA_CUTOFF = 7, 26
GROUPS = CHUNK // 64
QUARTER = CHUNK // 4

MAGIC = 12582912.0


def _sc_body(xyz_hbm, maskw_hbm, params_hbm, out_hbm,
             xyz_v, widx_v, aux_v, words_v, out_v, params_v, sem):
    wid = lax.axis_index("s") * 2 + lax.axis_index("c")

    pltpu.sync_copy(params_hbm, params_v)
    zero16 = lax.iota(jnp.int32, 16) * 0
    sx = params_v[pl.ds(0, 16)]
    sy = params_v[pl.ds(16, 16)]
    sz = params_v[pl.ds(32, 16)]
    hx = params_v[pl.ds(48, 16)]
    hy = params_v[pl.ds(64, 16)]
    hz = params_v[pl.ds(80, 16)]

    iota12 = lax.iota(jnp.int32, 16) * 12

    def do_chunk(i, _):
        c = wid + NW * i
        pltpu.sync_copy(xyz_hbm.at[pl.ds(c * (CHUNK * 3), CHUNK * 3)], xyz_v)

        def compute(g, _):
            for cc in range(4):
                px = iota12 + (192 * g + 3 * cc)
                x = plsc.load_gather(xyz_v, [px])
                y = plsc.load_gather(xyz_v, [px + 1])
                z = plsc.load_gather(xyz_v, [px + 2])
                fx = x * sx + hx
                fy = y * sy + hy
                fz = z * sz + hz
                rx = (fx + MAGIC) - MAGIC
                ry = (fy + MAGIC) - MAGIC
                rz = (fz + MAGIC) - MAGIC
                valid = ((rx >= 0.0) & (rx <= 255.0)
                         & (ry >= 0.0) & (ry <= 255.0)
                         & (rz >= 0.0) & (rz <= 255.0))
                ix = jnp.clip(rx, 0.0, 255.0).astype(jnp.int32)
                iy = jnp.clip(ry, 0.0, 255.0).astype(jnp.int32)
                iz = jnp.clip(rz, 0.0, 255.0).astype(jnp.int32)
                flat = (ix << 16) | (iy << 8) | iz
                shf = (flat & 3) << 3
                aux = shf | jnp.where(valid, 256, 0)
                off = cc * QUARTER + g * 16
                widx_v[pl.ds(off, 16)] = flat >> 2
                aux_v[pl.ds(off, 16)] = aux
            return 0

        lax.fori_loop(0, GROUPS, compute, 0)

        pltpu.async_copy(maskw_hbm.at[widx_v], words_v, sem).wait()

        def extract(g, _):
            acc = zero16
            for cc in range(4):
                off = cc * QUARTER + g * 16
                word = words_v[pl.ds(off, 16)]
                aux = aux_v[pl.ds(off, 16)]
                shf = aux & 31
                vbit = lax.shift_right_logical(aux, 8)
                bit = lax.shift_right_logical(word, shf) & 1 & vbit
                acc = acc | (bit << (8 * cc))
            out_v[pl.ds(g * 16, 16)] = acc
            return 0

        lax.fori_loop(0, GROUPS, extract, 0)
        pltpu.sync_copy(out_v, out_hbm.at[pl.ds(c * QUARTER, QUARTER)])
        return 0

    n_iters = BASE_ITERS + jnp.where(wid < EXTRA_CUTOFF, 1, 0)
    lax.fori_loop(0, n_iters, do_chunk, 0)


@jax.jit
def _sc_call(xyz_flat, maskw, params):
    mesh = plsc.VectorSubcoreMesh(core_axis_name="c", subcore_axis_name="s")
    return pl.kernel(
        _sc_body,
        out_type=jax.ShapeDtypeStruct((N_POINTS // 4,), jnp.int32),
        mesh=mesh,
        scratch_types=[
            pltpu.VMEM((CHUNK * 3,), jnp.float32),
            pltpu.VMEM((CHUNK,), jnp.int32),
            pltpu.VMEM((CHUNK,), jnp.int32),
            pltpu.VMEM((CHUNK,), jnp.int32),
            pltpu.VMEM((QUARTER,), jnp.int32),
            pltpu.VMEM((96,), jnp.float32),
            pltpu.SemaphoreType.DMA,
        ],
        compiler_params=pltpu.CompilerParams(needs_layout_passes=False),
    )(xyz_flat, maskw, params)


def kernel(xyz, mask, xyz2ijk_scale, xyz2ijk_shift):
    maskw = lax.bitcast_convert_type(
        mask.reshape(-1, 4).astype(jnp.uint8), jnp.int32)
    params = jnp.repeat(
        jnp.concatenate([xyz2ijk_scale.astype(jnp.float32),
                         xyz2ijk_shift.astype(jnp.float32)]), 16)
    out_w = _sc_call(xyz.reshape(-1), maskw, params)
    out_b = lax.bitcast_convert_type(out_w, jnp.uint8).reshape(-1)
    return out_b.astype(jnp.bool_)


# speedup vs baseline: 9.6875x; 9.6875x over previous
import jax
import jax.numpy as jnp
from jax import lax
from jax.experimental import pallas as pl
from jax.experimental.pallas import tpu as pltpu
from jax.experimental.pallas import tpu_sc as plsc

N_POINTS = 2_000_000
N_PAD = 2_000_896
CHUNK = 6_144
NW = 32
N_FULL = 325
BASE_ITERS, EXTRA_CUTOFF = 10, 5
TAIL_BASE = N_FULL * CHUNK
TAIL_OUT = N_PAD - TAIL_BASE
TAIL_IN = N_POINTS - TAIL_BASE

MAGIC = 12582912.0


def _sc_body(xs_hbm, ys_hbm, zs_hbm, maskw_hbm, params_hbm, out_hbm,
             xs_v, ys_v, zs_v, widx_v, aux_v, words_v, out_v, params_v, sem):
    wid = lax.axis_index("s") * 2 + lax.axis_index("c")

    pltpu.sync_copy(params_hbm, params_v)
    zero16 = lax.iota(jnp.int32, 16) * 0
    sx = params_v[pl.ds(0, 16)]
    sy = params_v[pl.ds(16, 16)]
    sz = params_v[pl.ds(32, 16)]
    hx = params_v[pl.ds(48, 16)]
    hy = params_v[pl.ds(64, 16)]
    hz = params_v[pl.ds(80, 16)]

    iota4 = lax.iota(jnp.int32, 16) * 4

    def process(base, base_w, npts_out, npts_in):
        g_in = npts_in // 64
        g_out = npts_out // 64
        quarter = npts_out // 4
        pltpu.sync_copy(xs_hbm.at[pl.ds(base, npts_in)],
                        xs_v.at[pl.ds(0, npts_in)])
        pltpu.sync_copy(ys_hbm.at[pl.ds(base, npts_in)],
                        ys_v.at[pl.ds(0, npts_in)])
        pltpu.sync_copy(zs_hbm.at[pl.ds(base, npts_in)],
                        zs_v.at[pl.ds(0, npts_in)])

        def compute(g, _):
            for cc in range(4):
                px = iota4 + (64 * g + cc)
                x = plsc.load_gather(xs_v, [px])
                y = plsc.load_gather(ys_v, [px])
                z = plsc.load_gather(zs_v, [px])
                fx = x * sx + hx
                fy = y * sy + hy
                fz = z * sz + hz
                rx = (fx + MAGIC) - MAGIC
                ry = (fy + MAGIC) - MAGIC
                rz = (fz + MAGIC) - MAGIC
                valid = ((rx >= 0.0) & (rx <= 255.0)
                         & (ry >= 0.0) & (ry <= 255.0)
                         & (rz >= 0.0) & (rz <= 255.0))
                ix = jnp.clip(rx, 0.0, 255.0).astype(jnp.int32)
                iy = jnp.clip(ry, 0.0, 255.0).astype(jnp.int32)
                iz = jnp.clip(rz, 0.0, 255.0).astype(jnp.int32)
                w = (ix << 14) | ((iy >> 2) << 8) | iz
                shf = (iy & 3) << 3
                aux = shf | jnp.where(valid, 256, 0)
                off = cc * quarter + g * 16
                widx_v[pl.ds(off, 16)] = w
                aux_v[pl.ds(off, 16)] = aux
            return 0

        lax.fori_loop(0, g_in, compute, 0)

        def zerofill(g, _):
            for cc in range(4):
                off = cc * quarter + g * 16
                widx_v[pl.ds(off, 16)] = zero16
                aux_v[pl.ds(off, 16)] = zero16
            return 0

        if npts_in < npts_out:
            lax.fori_loop(g_in, g_out, zerofill, 0)

        pltpu.async_copy(
            maskw_hbm.at[widx_v.at[pl.ds(0, npts_out)]],
            words_v.at[pl.ds(0, npts_out)], sem).wait()

        def extract(g, _):
            acc = zero16
            for cc in range(4):
                off = cc * quarter + g * 16
                word = words_v[pl.ds(off, 16)]
                aux = aux_v[pl.ds(off, 16)]
                shf = aux & 31
                vbit = lax.shift_right_logical(aux, 8)
                bit = lax.shift_right_logical(word, shf) & 1 & vbit
                acc = acc | (bit << (8 * cc))
            out_v[pl.ds(g * 16, 16)] = acc
            return 0

        lax.fori_loop(0, g_out, extract, 0)
        pltpu.sync_copy(out_v.at[pl.ds(0, quarter)],
                        out_hbm.at[pl.ds(base_w, quarter)])

    def do_chunk(i, _):
        c = wid + NW * i
        process(c * CHUNK, c * (CHUNK // 4), CHUNK, CHUNK)
        return 0

    n_iters = BASE_ITERS + jnp.where(wid < EXTRA_CUTOFF, 1, 0)
    lax.fori_loop(0, n_iters, do_chunk, 0)

    @pl.when(wid == 31)
    def _():
        process(TAIL_BASE, TAIL_BASE // 4, TAIL_OUT, TAIL_IN)


@jax.jit
def _sc_call(xs, ys, zs, maskw, params):
    mesh = plsc.VectorSubcoreMesh(core_axis_name="c", subcore_axis_name="s")
    return pl.kernel(
        _sc_body,
        out_type=jax.ShapeDtypeStruct((N_PAD // 4,), jnp.int32),
        mesh=mesh,
        scratch_types=[
            pltpu.VMEM((CHUNK,), jnp.float32),
            pltpu.VMEM((CHUNK,), jnp.float32),
            pltpu.VMEM((CHUNK,), jnp.float32),
            pltpu.VMEM((CHUNK,), jnp.int32),
            pltpu.VMEM((CHUNK,), jnp.int32),
            pltpu.VMEM((CHUNK,), jnp.int32),
            pltpu.VMEM((CHUNK // 4,), jnp.int32),
            pltpu.VMEM((96,), jnp.float32),
            pltpu.SemaphoreType.DMA,
        ],
        compiler_params=pltpu.CompilerParams(needs_layout_passes=False),
    )(xs, ys, zs, maskw, params)


def kernel(xyz, mask, xyz2ijk_scale, xyz2ijk_shift):
    xs = xyz[:, 0]
    ys = xyz[:, 1]
    zs = xyz[:, 2]
    m8 = mask.astype(jnp.uint8).reshape(256, 64, 4, 256)
    maskw = lax.bitcast_convert_type(
        m8.transpose(0, 1, 3, 2), jnp.int32).reshape(-1)
    params = jnp.repeat(
        jnp.concatenate([xyz2ijk_scale.astype(jnp.float32),
                         xyz2ijk_shift.astype(jnp.float32)]), 16)
    out_w = _sc_call(xs, ys, zs, maskw, params)
    out_u8 = lax.bitcast_convert_type(out_w, jnp.uint8).reshape(-1)
    return out_u8[:N_POINTS].astype(jnp.bool_)


# speedup vs baseline: 19.7916x; 2.0430x over previous
import jax
import jax.numpy as jnp
from jax import lax
from jax.experimental import pallas as pl
from jax.experimental.pallas import tpu as pltpu
from jax.experimental.pallas import tpu_sc as plsc

N_POINTS = 2_000_000
CHUNK = 8_000
NW = 32
BASE_ITERS, EXTRA_CUTOFF = 7, 26
GROUPS = CHUNK // 16

MAGIC = 12582912.0


def _sc_body(xs_hbm, ys_hbm, zs_hbm, maskw_hbm, params_hbm, out_hbm,
             xs_v, ys_v, zs_v, widx_v, aux_v, words_v, out_v, params_v, sem):
    wid = lax.axis_index("s") * 2 + lax.axis_index("c")

    pltpu.sync_copy(params_hbm, params_v)
    sx = params_v[pl.ds(0, 16)]
    sy = params_v[pl.ds(16, 16)]
    sz = params_v[pl.ds(32, 16)]
    hx = params_v[pl.ds(48, 16)]
    hy = params_v[pl.ds(64, 16)]
    hz = params_v[pl.ds(80, 16)]

    def do_chunk(i, _):
        c = wid + NW * i
        base = c * CHUNK
        pltpu.sync_copy(xs_hbm.at[pl.ds(base, CHUNK)], xs_v)
        pltpu.sync_copy(ys_hbm.at[pl.ds(base, CHUNK)], ys_v)
        pltpu.sync_copy(zs_hbm.at[pl.ds(base, CHUNK)], zs_v)

        def compute(g, _):
            off = g * 16
            x = xs_v[pl.ds(off, 16)]
            y = ys_v[pl.ds(off, 16)]
            z = zs_v[pl.ds(off, 16)]
            fx = x * sx + hx
            fy = y * sy + hy
            fz = z * sz + hz
            rx = (fx + MAGIC) - MAGIC
            ry = (fy + MAGIC) - MAGIC
            rz = (fz + MAGIC) - MAGIC
            valid = ((rx >= 0.0) & (rx <= 255.0)
                     & (ry >= 0.0) & (ry <= 255.0)
                     & (rz >= 0.0) & (rz <= 255.0))
            ix = jnp.clip(rx, 0.0, 255.0).astype(jnp.int32)
            iy = jnp.clip(ry, 0.0, 255.0).astype(jnp.int32)
            iz = jnp.clip(rz, 0.0, 255.0).astype(jnp.int32)
            w = (ix << 14) | ((iy >> 2) << 8) | iz
            shf = (iy & 3) << 3
            aux = shf | jnp.where(valid, 256, 0)
            widx_v[pl.ds(off, 16)] = w
            aux_v[pl.ds(off, 16)] = aux
            return 0

        lax.fori_loop(0, GROUPS, compute, 0)

        pltpu.async_copy(maskw_hbm.at[widx_v], words_v, sem).wait()

        def extract(g, _):
            off = g * 16
            word = words_v[pl.ds(off, 16)]
            aux = aux_v[pl.ds(off, 16)]
            shf = aux & 31
            vbit = lax.shift_right_logical(aux, 8)
            out_v[pl.ds(off, 16)] = lax.shift_right_logical(word, shf) & 1 & vbit
            return 0

        lax.fori_loop(0, GROUPS, extract, 0)
        pltpu.sync_copy(out_v, out_hbm.at[pl.ds(base, CHUNK)])
        return 0

    n_iters = BASE_ITERS + jnp.where(wid < EXTRA_CUTOFF, 1, 0)
    lax.fori_loop(0, n_iters, do_chunk, 0)


@jax.jit
def _sc_call(xs, ys, zs, maskw, params):
    mesh = plsc.VectorSubcoreMesh(core_axis_name="c", subcore_axis_name="s")
    return pl.kernel(
        _sc_body,
        out_type=jax.ShapeDtypeStruct((N_POINTS,), jnp.int32),
        mesh=mesh,
        scratch_types=[
            pltpu.VMEM((CHUNK,), jnp.float32),
            pltpu.VMEM((CHUNK,), jnp.float32),
            pltpu.VMEM((CHUNK,), jnp.float32),
            pltpu.VMEM((CHUNK,), jnp.int32),
            pltpu.VMEM((CHUNK,), jnp.int32),
            pltpu.VMEM((CHUNK,), jnp.int32),
            pltpu.VMEM((CHUNK,), jnp.int32),
            pltpu.VMEM((96,), jnp.float32),
            pltpu.SemaphoreType.DMA,
        ],
        compiler_params=pltpu.CompilerParams(needs_layout_passes=False),
    )(xs, ys, zs, maskw, params)


def kernel(xyz, mask, xyz2ijk_scale, xyz2ijk_shift):
    xs = xyz[:, 0]
    ys = xyz[:, 1]
    zs = xyz[:, 2]
    m = mask.reshape(256, 64, 4, 256)
    maskw = (m[:, :, 0, :].astype(jnp.int32)
             | (m[:, :, 1, :].astype(jnp.int32) << 8)
             | (m[:, :, 2, :].astype(jnp.int32) << 16)
             | (m[:, :, 3, :].astype(jnp.int32) << 24)).reshape(-1)
    params = jnp.repeat(
        jnp.concatenate([xyz2ijk_scale.astype(jnp.float32),
                         xyz2ijk_shift.astype(jnp.float32)]), 16)
    out_w = _sc_call(xs, ys, zs, maskw, params)
    return out_w.astype(jnp.bool_)
